# Initial kernel scaffold; baseline (speedup 1.0000x reference)
import jax
import jax.numpy as jnp
from jax.experimental import pallas as pl


def kernel(x, W_proj, b_proj, W_head, b_head):
    raise NotImplementedError("write your pallas kernel here")


# speedup vs baseline: 1.7555x; 1.7555x over previous
import functools

import jax
import jax.numpy as jnp
from jax import lax
from jax.experimental import pallas as pl
from jax.experimental.pallas import tpu as pltpu
from jax.experimental.pallas import tpu_sc as plsc

B = 1024
V = 100000
D = 128
C = 5000
L = 100

LANES = 16
LP = 112
NV = LP // LANES
NC, NS = 2, 16
NW = NC * NS
RPT = B // NW
ND = D // LANES
MARK_N = 100008


def _sc_embed_body(x_hbm, table_hbm, bp_hbm, out_hbm,
                   xt, cidx, mark, rows, accv, bpv, sem):
    wid = lax.axis_index("s") * NC + lax.axis_index("c")
    base = wid * RPT

    pltpu.sync_copy(x_hbm.at[pl.ds(base, RPT)], xt)
    pltpu.sync_copy(bp_hbm, bpv)

    def dedup_row(r, carry):
        for v in range(NV):
            idxv = xt[r, pl.ds(v * LANES, LANES)]
            lane = jnp.int32(v * LANES) + lax.iota(jnp.int32, LANES)
            plsc.store_scatter(mark, [idxv], lane)
        for v in range(NV):
            idxv = xt[r, pl.ds(v * LANES, LANES)]
            lane = jnp.int32(v * LANES) + lax.iota(jnp.int32, LANES)
            g = plsc.load_gather(mark, [idxv])
            keep = (g == lane) & (idxv != jnp.int32(V))
            cidx[r, pl.ds(v * LANES, LANES)] = jnp.where(
                keep, idxv, jnp.int32(V))
        return carry

    lax.fori_loop(0, RPT, dedup_row, 0)

    def row_gather(r, carry):
        pltpu.async_copy(table_hbm.at[cidx.at[r]], rows, sem).wait()
        for d8 in range(ND):
            accv[r, pl.ds(d8 * LANES, LANES)] = bpv[pl.ds(d8 * LANES, LANES)]

        def add_one(j, c2):
            for d8 in range(ND):
                plsc.addupdate(accv.at[r, pl.ds(d8 * LANES, LANES)],
                               rows[j, pl.ds(d8 * LANES, LANES)])
            return c2

        lax.fori_loop(0, LP, add_one, 0)
        return carry

    lax.fori_loop(0, RPT, row_gather, 0)
    pltpu.sync_copy(accv, out_hbm.at[pl.ds(base, RPT)])


def _sc_embed(x_pad, table, b_proj):
    mesh = plsc.VectorSubcoreMesh(core_axis_name="c", subcore_axis_name="s")
    fn = pl.kernel(
        _sc_embed_body,
        mesh=mesh,
        compiler_params=pltpu.CompilerParams(needs_layout_passes=False),
        out_type=jax.ShapeDtypeStruct((B, D), jnp.float32),
        scratch_types=[
            pltpu.VMEM((RPT, LP), jnp.int32),
            pltpu.VMEM((RPT, LP), jnp.int32),
            pltpu.VMEM((MARK_N,), jnp.int32),
            pltpu.VMEM((LP, D), jnp.float32),
            pltpu.VMEM((RPT, D), jnp.float32),
            pltpu.VMEM((D,), jnp.float32),
            pltpu.SemaphoreType.DMA,
        ],
    )
    return fn(x_pad, table, b_proj)


BB = 256
BC = 512


def _head_mm_body(xp_ref, wh_ref, bh_ref, o_ref):
    acc = jax.lax.dot_general(
        xp_ref[...], wh_ref[...], (((1,), (0,)), ((), ())),
        preferred_element_type=jnp.float32)
    o_ref[...] = acc + bh_ref[...]


def _head_mm(x_proj, wh_t, b_head):
    grid = (B // BB, pl.cdiv(C, BC))
    return pl.pallas_call(
        _head_mm_body,
        grid=grid,
        in_specs=[
            pl.BlockSpec((BB, D), lambda i, j: (i, 0)),
            pl.BlockSpec((D, BC), lambda i, j: (0, j)),
            pl.BlockSpec((1, BC), lambda i, j: (0, j)),
        ],
        out_specs=pl.BlockSpec((BB, BC), lambda i, j: (i, j)),
        out_shape=jax.ShapeDtypeStruct((B, C), jnp.float32),
    )(x_proj, wh_t, b_head.reshape(1, C))


def kernel(x, W_proj, b_proj, W_head, b_head):
    x_pad = jnp.pad(x.astype(jnp.int32), ((0, 0), (0, LP - L)),
                    constant_values=V)
    table = jnp.concatenate(
        [W_proj.T, jnp.zeros((1, D), jnp.float32)], axis=0)
    x_proj = _sc_embed(x_pad, table, b_proj)
    return _head_mm(x_proj, W_head.T, b_head)


# speedup vs baseline: 1.7809x; 1.0145x over previous
import jax
import jax.numpy as jnp
from jax import lax
from jax.experimental import pallas as pl
from jax.experimental.pallas import tpu as pltpu
from jax.experimental.pallas import tpu_sc as plsc

B = 1024
V = 100000
D = 128
C = 5000
L = 100

LANES = 16
LP = 112
NV = LP // LANES
HC = LP // 2
NC, NS = 2, 16
NW = NC * NS
RPT = B // NW
ND = D // LANES
MARK_N = 100008


def _sc_embed_body(x_hbm, table_hbm, out_hbm, xt, mark, bufs, accv,
                   sem0, sem1):
    wid = lax.axis_index("s") * NC + lax.axis_index("c")
    base = wid * RPT
    sems = (sem0, sem1)

    pltpu.sync_copy(x_hbm.at[pl.ds(base, RPT)], xt)

    def dedup_row(r):
        for v in range(NV):
            idxv = xt[r, pl.ds(v * LANES, LANES)]
            lane = jnp.int32(v * LANES) + lax.iota(jnp.int32, LANES)
            plsc.store_scatter(mark, [idxv], lane)
        for v in range(NV):
            idxv = xt[r, pl.ds(v * LANES, LANES)]
            lane = jnp.int32(v * LANES) + lax.iota(jnp.int32, LANES)
            g = plsc.load_gather(mark, [idxv])
            keep = (g == lane) & (idxv != jnp.int32(V))
            xt[r, pl.ds(v * LANES, LANES)] = jnp.where(
                keep, idxv, jnp.int32(V))

    def start_chunk(r, half):
        pltpu.async_copy(
            table_hbm.at[xt.at[r, pl.ds(half * HC, HC)]],
            bufs.at[half], sems[half])

    dedup_row(0)
    start_chunk(0, 0)
    start_chunk(0, 1)

    def body(r, carry):
        @pl.when(r + 1 < RPT)
        def _():
            dedup_row(r + 1)

        acc = tuple(jnp.zeros((LANES,), jnp.float32) for _ in range(ND))
        for half in range(2):
            pltpu.make_async_copy(
                table_hbm.at[xt.at[r, pl.ds(half * HC, HC)]],
                bufs.at[half], sems[half]).wait()

            def add_j(j, a, _half=half):
                return tuple(
                    a[d] + bufs[_half, j, pl.ds(d * LANES, LANES)]
                    for d in range(ND))

            acc = lax.fori_loop(0, HC, add_j, acc)

            @pl.when(r + 1 < RPT)
            def _(_half=half):
                start_chunk(r + 1, _half)

        for d in range(ND):
            accv[r, pl.ds(d * LANES, LANES)] = acc[d]
        return carry

    lax.fori_loop(0, RPT, body, 0)
    pltpu.sync_copy(accv, out_hbm.at[pl.ds(base, RPT)])


def _sc_embed(x_pad, table):
    mesh = plsc.VectorSubcoreMesh(core_axis_name="c", subcore_axis_name="s")
    fn = pl.kernel(
        _sc_embed_body,
        mesh=mesh,
        compiler_params=pltpu.CompilerParams(needs_layout_passes=False),
        out_type=jax.ShapeDtypeStruct((B, D), jnp.float32),
        scratch_types=[
            pltpu.VMEM((RPT, LP), jnp.int32),
            pltpu.VMEM((MARK_N,), jnp.int32),
            pltpu.VMEM((2, HC, D), jnp.float32),
            pltpu.VMEM((RPT, D), jnp.float32),
            pltpu.SemaphoreType.DMA,
            pltpu.SemaphoreType.DMA,
        ],
    )
    return fn(x_pad, table)


BB = 256
BC = 512


def _head_mm_body(xp_ref, bp_ref, wh_ref, bh_ref, o_ref):
    acc = jax.lax.dot_general(
        xp_ref[...] + bp_ref[...], wh_ref[...], (((1,), (0,)), ((), ())),
        preferred_element_type=jnp.float32)
    o_ref[...] = acc + bh_ref[...]


def _head_mm(x_proj, b_proj, wh_t, b_head):
    grid = (B // BB, pl.cdiv(C, BC))
    return pl.pallas_call(
        _head_mm_body,
        grid=grid,
        in_specs=[
            pl.BlockSpec((BB, D), lambda i, j: (i, 0)),
            pl.BlockSpec((1, D), lambda i, j: (0, 0)),
            pl.BlockSpec((D, BC), lambda i, j: (0, j)),
            pl.BlockSpec((1, BC), lambda i, j: (0, j)),
        ],
        out_specs=pl.BlockSpec((BB, BC), lambda i, j: (i, j)),
        out_shape=jax.ShapeDtypeStruct((B, C), jnp.float32),
    )(x_proj, b_proj.reshape(1, D), wh_t, b_head.reshape(1, C))


def kernel(x, W_proj, b_proj, W_head, b_head):
    x_pad = jnp.pad(x.astype(jnp.int32), ((0, 0), (0, LP - L)),
                    constant_values=V)
    table = jnp.concatenate(
        [W_proj.T, jnp.zeros((1, D), jnp.float32)], axis=0)
    x_proj = _sc_embed(x_pad, table)
    return _head_mm(x_proj, b_proj, W_head.T, b_head)


# speedup vs baseline: 3.0444x; 1.7095x over previous
import jax
import jax.numpy as jnp
from jax import lax
from jax.experimental import pallas as pl
from jax.experimental.pallas import tpu as pltpu
from jax.experimental.pallas import tpu_sc as plsc

B = 1024
V = 100000
D = 128
C = 5000
L = 100

LANES = 16
LP = 112
NV = LP // LANES
NC, NS = 2, 16
NW = NC * NS
RPT = B // NW
ND = D // LANES
CH = 5504
NCHUNK = 19
TPAD = NCHUNK * CH
SLICE = CH // NS
GCAP = 144


def _sc_embed_body(x_hbm, table_hbm, out_hbm,
                   xt, mark, glist, gidx, grows, accv,
                   sbuf0, sbuf1, semst0, semst1, semg):
    cid = lax.axis_index("c")
    sid = lax.axis_index("s")
    wid = sid * NC + cid
    base = wid * RPT
    sbufs = (sbuf0, sbuf1)
    semsts = (semst0, semst1)

    pltpu.sync_copy(x_hbm.at[pl.ds(base, RPT)], xt)

    zf = jnp.zeros((LANES,), jnp.float32)

    def zero_row(r, carry):
        for d8 in range(ND):
            accv[r, pl.ds(d8 * LANES, LANES)] = zf
        return carry

    lax.fori_loop(0, RPT, zero_row, 0)

    def issue_stage(c):
        sb, sem = sbufs[c % 2], semsts[c % 2]
        row0 = c * CH + sid * SLICE
        pltpu.async_copy(table_hbm.at[pl.ds(row0, SLICE)],
                         sb.at[pl.ds(sid * SLICE, SLICE)], sem)

    def wait_stage(c):
        sb, sem = sbufs[c % 2], semsts[c % 2]
        row0 = c * CH + sid * SLICE
        pltpu.make_async_copy(table_hbm.at[pl.ds(row0, SLICE)],
                              sb.at[pl.ds(sid * SLICE, SLICE)], sem).wait()

    issue_stage(0)
    issue_stage(1)

    for c in range(NCHUNK):
        sb = sbufs[c % 2]
        lo = jnp.int32(c * CH)

        wait_stage(c)
        plsc.subcore_barrier()

        def row_proc(r, carry, _sb=sb, _lo=lo):
            zi = jnp.zeros((LANES,), jnp.int32)
            for q in range(8):
                glist[pl.ds(q * LANES, LANES)] = zi
            lidxs, masks, lanes = [], [], []
            for v in range(NV):
                idxv = xt[r, pl.ds(v * LANES, LANES)]
                lidx = idxv - _lo
                m = (lidx >= 0) & (lidx < jnp.int32(CH))
                lane = jnp.int32(v * LANES) + lax.iota(jnp.int32, LANES)
                plsc.store_scatter(mark, [lidx], lane, mask=m)
                lidxs.append(lidx)
                masks.append(m)
                lanes.append(lane)
            off = jnp.zeros((LANES,), jnp.int32)
            for v in range(NV):
                g = plsc.load_gather(mark, [lidxs[v]], mask=masks[v])
                keep = masks[v] & (g == lanes[v])
                cum = jnp.cumsum(keep.astype(jnp.int32))
                dest = off + cum - 1
                plsc.store_scatter(glist, [dest], lidxs[v], mask=keep)
                off = off + plsc.all_reduce_population_count(keep)
            k = off[0]
            n16 = (k + 15) >> 4

            def fire(g2, c2):
                gidx[...] = glist[pl.ds(g2 * LANES, LANES)]
                pltpu.async_copy(
                    _sb.at[gidx],
                    grows.at[pl.ds(g2 * LANES, LANES)], semg).wait()
                return c2

            lax.fori_loop(0, n16, fire, 0)

            acc = tuple(accv[r, pl.ds(d8 * LANES, LANES)]
                        for d8 in range(ND))

            def add_j(j, a):
                return tuple(a[d8] + grows[j, pl.ds(d8 * LANES, LANES)]
                             for d8 in range(ND))

            acc = lax.fori_loop(0, k, add_j, acc)
            for d8 in range(ND):
                accv[r, pl.ds(d8 * LANES, LANES)] = acc[d8]
            return carry

        lax.fori_loop(0, RPT, row_proc, 0)
        plsc.subcore_barrier()
        if c + 2 < NCHUNK:
            issue_stage(c + 2)

    pltpu.sync_copy(accv, out_hbm.at[pl.ds(base, RPT)])


def _sc_embed(x_pad, table):
    mesh = plsc.VectorSubcoreMesh(core_axis_name="c", subcore_axis_name="s")
    fn = pl.kernel(
        _sc_embed_body,
        mesh=mesh,
        compiler_params=pltpu.CompilerParams(needs_layout_passes=False),
        out_type=jax.ShapeDtypeStruct((B, D), jnp.float32),
        scratch_types=[
            pltpu.VMEM((RPT, LP), jnp.int32),
            pltpu.VMEM((CH,), jnp.int32),
            pltpu.VMEM((GCAP,), jnp.int32),
            pltpu.VMEM((LANES,), jnp.int32),
            pltpu.VMEM((GCAP, D), jnp.float32),
            pltpu.VMEM((RPT, D), jnp.float32),
            pltpu.VMEM_SHARED((CH, D), jnp.float32),
            pltpu.VMEM_SHARED((CH, D), jnp.float32),
            pltpu.SemaphoreType.DMA,
            pltpu.SemaphoreType.DMA,
            pltpu.SemaphoreType.DMA,
        ],
    )
    return fn(x_pad, table)


BB = 256
BC = 512


def _head_mm_body(xp_ref, bp_ref, wh_ref, bh_ref, o_ref):
    acc = jax.lax.dot_general(
        xp_ref[...] + bp_ref[...], wh_ref[...], (((1,), (0,)), ((), ())),
        preferred_element_type=jnp.float32)
    o_ref[...] = acc + bh_ref[...]


def _head_mm(x_proj, b_proj, wh_t, b_head):
    grid = (B // BB, pl.cdiv(C, BC))
    return pl.pallas_call(
        _head_mm_body,
        grid=grid,
        in_specs=[
            pl.BlockSpec((BB, D), lambda i, j: (i, 0)),
            pl.BlockSpec((1, D), lambda i, j: (0, 0)),
            pl.BlockSpec((D, BC), lambda i, j: (0, j)),
            pl.BlockSpec((1, BC), lambda i, j: (0, j)),
        ],
        out_specs=pl.BlockSpec((BB, BC), lambda i, j: (i, j)),
        out_shape=jax.ShapeDtypeStruct((B, C), jnp.float32),
    )(x_proj, b_proj.reshape(1, D), wh_t, b_head.reshape(1, C))


def kernel(x, W_proj, b_proj, W_head, b_head):
    x_pad = jnp.pad(x.astype(jnp.int32), ((0, 0), (0, LP - L)),
                    constant_values=V)
    table = jnp.concatenate(
        [W_proj.T, jnp.zeros((TPAD - V, D), jnp.float32)], axis=0)
    x_proj = _sc_embed(x_pad, table)
    return _head_mm(x_proj, b_proj, W_head.T, b_head)


# speedup vs baseline: 4.1961x; 1.3783x over previous
import jax
import jax.numpy as jnp
from jax import lax
from jax.experimental import pallas as pl
from jax.experimental.pallas import tpu as pltpu
from jax.experimental.pallas import tpu_sc as plsc

B = 1024
V = 100000
D = 128
C = 5000
L = 100

LANES = 16
LP = 112
NV = LP // LANES
NC, NS = 2, 16
NW = NC * NS
RPT = B // NW
ND = D // LANES
CH = 5376
NCHUNK = 19
TPAD = NCHUNK * CH
SLICE = CH // NS
GCAP = 144


def _sc_embed_body(x_hbm, table_hbm, out_hbm,
                   xt, mark, glist0, glist1, grows0, grows1, accv,
                   sbuf0, sbuf1, semst0, semst1, semg0, semg1):
    cid = lax.axis_index("c")
    sid = lax.axis_index("s")
    wid = sid * NC + cid
    base = wid * RPT
    sbufs = (sbuf0, sbuf1)
    semsts = (semst0, semst1)

    pltpu.sync_copy(x_hbm.at[pl.ds(base, RPT)], xt)

    zf = jnp.zeros((LANES,), jnp.float32)

    def zero_row(r, carry):
        for d8 in range(ND):
            accv[r, pl.ds(d8 * LANES, LANES)] = zf
        return carry

    lax.fori_loop(0, RPT, zero_row, 0)

    def issue_stage(c, par):
        sb, sem = sbufs[par], semsts[par]
        row0 = c * CH + sid * SLICE
        pltpu.async_copy(table_hbm.at[pl.ds(row0, SLICE)],
                         sb.at[pl.ds(sid * SLICE, SLICE)], sem)

    def wait_stage(c, par):
        sb, sem = sbufs[par], semsts[par]
        row0 = c * CH + sid * SLICE
        pltpu.make_async_copy(table_hbm.at[pl.ds(row0, SLICE)],
                              sb.at[pl.ds(sid * SLICE, SLICE)], sem).wait()

    issue_stage(0, 0)
    issue_stage(1, 1)

    def chunk_body(c, par):
        sb = sbufs[par]
        lo = c * CH

        wait_stage(c, par)
        plsc.subcore_barrier()

        glists = (glist0, glist1)
        growss = (grows0, grows1)
        semgs = (semg0, semg1)

        def filter_fire(r, par, _sb=sb, _lo=lo):
            gl, gr, sg = glists[par], growss[par], semgs[par]
            zi = jnp.zeros((LANES,), jnp.int32)
            for q in range(8):
                gl[q, pl.ds(0, LANES)] = zi
            lidxs, masks, lanes = [], [], []
            for v in range(NV):
                idxv = xt[r, pl.ds(v * LANES, LANES)]
                lidx = idxv - _lo
                m = (lidx >= 0) & (lidx < jnp.int32(CH))
                lane = jnp.int32(v * LANES) + lax.iota(jnp.int32, LANES)
                plsc.store_scatter(mark, [lidx], lane, mask=m)
                lidxs.append(lidx)
                masks.append(m)
                lanes.append(lane)
            off = jnp.zeros((LANES,), jnp.int32)
            for v in range(NV):
                g = plsc.load_gather(mark, [lidxs[v]], mask=masks[v])
                keep = masks[v] & (g == lanes[v])
                cum = jnp.cumsum(keep.astype(jnp.int32))
                dest = off + cum - 1
                plsc.store_scatter(
                    gl, [jnp.right_shift(dest, 4), dest & jnp.int32(15)],
                    lidxs[v], mask=keep)
                off = off + plsc.all_reduce_population_count(keep)
            k = off[0]
            n16 = (k + 15) >> 4

            def fire(g2, c2):
                pltpu.async_copy(
                    _sb.at[gl.at[g2]],
                    gr.at[pl.ds(g2 * LANES, LANES)], sg)
                return c2

            lax.fori_loop(0, n16, fire, 0)
            return k

        def drain(par, k, _sb=sb):
            gl, gr, sg = glists[par], growss[par], semgs[par]
            n16 = (k + 15) >> 4

            def one(g2, c2):
                pltpu.make_async_copy(
                    _sb.at[gl.at[g2]],
                    gr.at[pl.ds(g2 * LANES, LANES)], sg).wait()
                return c2

            lax.fori_loop(0, n16, one, 0)

        def drain_acc(r, par, k):
            drain(par, k)
            gr = growss[par]
            acc = tuple(accv[r, pl.ds(d8 * LANES, LANES)]
                        for d8 in range(ND))

            def add_j(j, a):
                return tuple(a[d8] + gr[j, pl.ds(d8 * LANES, LANES)]
                             for d8 in range(ND))

            acc = lax.fori_loop(0, k, add_j, acc)
            for d8 in range(ND):
                accv[r, pl.ds(d8 * LANES, LANES)] = acc[d8]

        k_even = filter_fire(0, 0)

        def pair(t, kc):
            r0 = 2 * t
            k_odd = filter_fire(r0 + 1, 1)
            drain_acc(r0, 0, kc)
            k_next = filter_fire(jnp.minimum(r0 + 2, RPT - 1), 0)
            drain_acc(r0 + 1, 1, k_odd)
            return k_next

        k_left = lax.fori_loop(0, RPT // 2, pair, k_even)
        drain(0, k_left)
        plsc.subcore_barrier()

        @pl.when(c + 2 < NCHUNK)
        def _():
            issue_stage(c + 2, par)

    def chunk_pair(u, carry):
        chunk_body(2 * u, 0)
        chunk_body(2 * u + 1, 1)
        return carry

    lax.fori_loop(0, NCHUNK // 2, chunk_pair, 0)
    if NCHUNK % 2:
        chunk_body(jnp.int32(NCHUNK - 1), 0)

    pltpu.sync_copy(accv, out_hbm.at[pl.ds(base, RPT)])


def _sc_embed(x_pad, table):
    mesh = plsc.VectorSubcoreMesh(core_axis_name="c", subcore_axis_name="s")
    fn = pl.kernel(
        _sc_embed_body,
        mesh=mesh,
        compiler_params=pltpu.CompilerParams(needs_layout_passes=False),
        out_type=jax.ShapeDtypeStruct((B, D), jnp.float32),
        scratch_types=[
            pltpu.VMEM((RPT, LP), jnp.int32),
            pltpu.VMEM((CH,), jnp.int32),
            pltpu.VMEM((LP // LANES + 1, LANES), jnp.int32),
            pltpu.VMEM((LP // LANES + 1, LANES), jnp.int32),
            pltpu.VMEM((LP, D), jnp.float32),
            pltpu.VMEM((LP, D), jnp.float32),
            pltpu.VMEM((RPT, D), jnp.float32),
            pltpu.VMEM_SHARED((CH, D), jnp.float32),
            pltpu.VMEM_SHARED((CH, D), jnp.float32),
            pltpu.SemaphoreType.DMA,
            pltpu.SemaphoreType.DMA,
            pltpu.SemaphoreType.DMA,
            pltpu.SemaphoreType.DMA,
        ],
    )
    return fn(x_pad, table)


BB = 256
BC = 512


def _head_mm_body(xp_ref, bp_ref, wh_ref, bh_ref, o_ref):
    acc = jax.lax.dot_general(
        xp_ref[...] + bp_ref[...], wh_ref[...], (((1,), (0,)), ((), ())),
        preferred_element_type=jnp.float32)
    o_ref[...] = acc + bh_ref[...]


def _head_mm(x_proj, b_proj, wh_t, b_head):
    grid = (B // BB, pl.cdiv(C, BC))
    return pl.pallas_call(
        _head_mm_body,
        grid=grid,
        in_specs=[
            pl.BlockSpec((BB, D), lambda i, j: (i, 0)),
            pl.BlockSpec((1, D), lambda i, j: (0, 0)),
            pl.BlockSpec((D, BC), lambda i, j: (0, j)),
            pl.BlockSpec((1, BC), lambda i, j: (0, j)),
        ],
        out_specs=pl.BlockSpec((BB, BC), lambda i, j: (i, j)),
        out_shape=jax.ShapeDtypeStruct((B, C), jnp.float32),
    )(x_proj, b_proj.reshape(1, D), wh_t, b_head.reshape(1, C))


def kernel(x, W_proj, b_proj, W_head, b_head):
    x_pad = jnp.pad(x.astype(jnp.int32), ((0, 0), (0, LP - L)),
                    constant_values=V)
    table = jnp.concatenate(
        [W_proj.T, jnp.zeros((TPAD - V, D), jnp.float32)], axis=0)
    x_proj = _sc_embed(x_pad, table)
    return _head_mm(x_proj, b_proj, W_head.T, b_head)


# speedup vs baseline: 4.5136x; 1.0757x over previous
import jax
import jax.numpy as jnp
from jax import lax
from jax.experimental import pallas as pl
from jax.experimental.pallas import tpu as pltpu
from jax.experimental.pallas import tpu_sc as plsc

B = 1024
V = 100000
D = 128
C = 5000
L = 100

LANES = 16
LP = 112
NV = LP // LANES
NC, NS = 2, 16
NW = NC * NS
RPT = B // NW
ND = D // LANES
CH = 5376
NCHUNK = 19
TPAD = NCHUNK * CH
SLICE = CH // NS
GCAP = 144


def _sc_embed_body(x_hbm, table_hbm, out_hbm,
                   xt, mark, glist0, glist1, grows0, grows1, accv,
                   sbuf0, sbuf1, semst0, semst1, semg0, semg1):
    cid = lax.axis_index("c")
    sid = lax.axis_index("s")
    wid = sid * NC + cid
    base = wid * RPT
    sbufs = (sbuf0, sbuf1)
    semsts = (semst0, semst1)

    pltpu.sync_copy(x_hbm.at[pl.ds(base, RPT)], xt)

    zf = jnp.zeros((LANES,), jnp.float32)

    def zero_row(r, carry):
        for d8 in range(ND):
            accv[r, pl.ds(d8 * LANES, LANES)] = zf
        return carry

    lax.fori_loop(0, RPT, zero_row, 0)

    def issue_stage(c, par):
        sb, sem = sbufs[par], semsts[par]
        row0 = c * CH + sid * SLICE
        pltpu.async_copy(table_hbm.at[pl.ds(row0, SLICE)],
                         sb.at[pl.ds(sid * SLICE, SLICE)], sem)

    def wait_stage(c, par):
        sb, sem = sbufs[par], semsts[par]
        row0 = c * CH + sid * SLICE
        pltpu.make_async_copy(table_hbm.at[pl.ds(row0, SLICE)],
                              sb.at[pl.ds(sid * SLICE, SLICE)], sem).wait()

    issue_stage(0, 0)
    issue_stage(1, 1)

    def chunk_body(c, par):
        sb = sbufs[par]
        lo = c * CH

        wait_stage(c, par)
        plsc.subcore_barrier()

        glists = (glist0, glist1)
        growss = (grows0, grows1)
        semgs = (semg0, semg1)

        def filter_fire(r, par, _sb=sb, _lo=lo):
            gl, gr, sg = glists[par], growss[par], semgs[par]
            zi = jnp.zeros((LANES,), jnp.int32)
            for q in range(8):
                gl[q, pl.ds(0, LANES)] = zi
            lidxs, masks, lanes = [], [], []
            for v in range(NV):
                idxv = xt[r, pl.ds(v * LANES, LANES)]
                lidx = idxv - _lo
                m = (lidx >= 0) & (lidx < jnp.int32(CH))
                lane = jnp.int32(v * LANES) + lax.iota(jnp.int32, LANES)
                plsc.store_scatter(mark, [lidx], lane, mask=m)
                lidxs.append(lidx)
                masks.append(m)
                lanes.append(lane)
            off = jnp.zeros((LANES,), jnp.int32)
            for v in range(NV):
                g = plsc.load_gather(mark, [lidxs[v]], mask=masks[v])
                keep = masks[v] & (g == lanes[v])
                cum = jnp.cumsum(keep.astype(jnp.int32))
                dest = off + cum - 1
                plsc.store_scatter(
                    gl, [jnp.right_shift(dest, 4), dest & jnp.int32(15)],
                    lidxs[v], mask=keep)
                off = off + plsc.all_reduce_population_count(keep)
            k = off[0]
            n16 = (k + 15) >> 4

            def fire(g2, c2):
                pltpu.async_copy(
                    _sb.at[gl.at[g2]],
                    gr.at[pl.ds(g2 * LANES, LANES)], sg)
                return c2

            lax.fori_loop(0, n16, fire, 0)
            return k

        def drain(par, k, _sb=sb):
            gl, gr, sg = glists[par], growss[par], semgs[par]
            n16 = (k + 15) >> 4

            def one(g2, c2):
                pltpu.make_async_copy(
                    _sb.at[gl.at[g2]],
                    gr.at[pl.ds(g2 * LANES, LANES)], sg).wait()
                return c2

            lax.fori_loop(0, n16, one, 0)

        def drain_acc(r, par, k):
            drain(par, k)
            gr = growss[par]
            acc = tuple(accv[r, pl.ds(d8 * LANES, LANES)]
                        for d8 in range(ND))

            def add_j(j, a):
                return tuple(a[d8] + gr[j, pl.ds(d8 * LANES, LANES)]
                             for d8 in range(ND))

            acc = lax.fori_loop(0, k, add_j, acc)
            for d8 in range(ND):
                accv[r, pl.ds(d8 * LANES, LANES)] = acc[d8]

        k_even = filter_fire(0, 0)

        def pair(t, kc):
            r0 = 2 * t
            k_odd = filter_fire(r0 + 1, 1)
            drain_acc(r0, 0, kc)
            k_next = filter_fire(jnp.minimum(r0 + 2, RPT - 1), 0)
            drain_acc(r0 + 1, 1, k_odd)
            return k_next

        k_left = lax.fori_loop(0, RPT // 2, pair, k_even)
        drain(0, k_left)
        plsc.subcore_barrier()

        @pl.when(c + 2 < NCHUNK)
        def _():
            issue_stage(c + 2, par)

    def chunk_pair(u, carry):
        chunk_body(2 * u, 0)
        chunk_body(2 * u + 1, 1)
        return carry

    lax.fori_loop(0, NCHUNK // 2, chunk_pair, 0)
    if NCHUNK % 2:
        chunk_body(jnp.int32(NCHUNK - 1), 0)

    pltpu.sync_copy(accv, out_hbm.at[pl.ds(base, RPT)])


def _sc_embed(x_pad, table):
    mesh = plsc.VectorSubcoreMesh(core_axis_name="c", subcore_axis_name="s")
    fn = pl.kernel(
        _sc_embed_body,
        mesh=mesh,
        compiler_params=pltpu.CompilerParams(needs_layout_passes=False),
        out_type=jax.ShapeDtypeStruct((B, D), jnp.float32),
        scratch_types=[
            pltpu.VMEM((RPT, LP), jnp.int32),
            pltpu.VMEM((CH,), jnp.int32),
            pltpu.VMEM((LP // LANES + 1, LANES), jnp.int32),
            pltpu.VMEM((LP // LANES + 1, LANES), jnp.int32),
            pltpu.VMEM((LP, D), jnp.float32),
            pltpu.VMEM((LP, D), jnp.float32),
            pltpu.VMEM((RPT, D), jnp.float32),
            pltpu.VMEM_SHARED((CH, D), jnp.float32),
            pltpu.VMEM_SHARED((CH, D), jnp.float32),
            pltpu.SemaphoreType.DMA,
            pltpu.SemaphoreType.DMA,
            pltpu.SemaphoreType.DMA,
            pltpu.SemaphoreType.DMA,
        ],
    )
    return fn(x_pad, table)


BB = 512
BC = 1024


def _head_mm_body(xp_ref, bp_ref, wh_ref, bh_ref, o_ref):
    acc = jax.lax.dot_general(
        xp_ref[...] + bp_ref[...], wh_ref[...], (((1,), (0,)), ((), ())),
        preferred_element_type=jnp.float32)
    o_ref[...] = acc + bh_ref[...]


def _head_mm(x_proj, b_proj, wh_t, b_head):
    grid = (B // BB, pl.cdiv(C, BC))
    return pl.pallas_call(
        _head_mm_body,
        grid=grid,
        in_specs=[
            pl.BlockSpec((BB, D), lambda i, j: (i, 0)),
            pl.BlockSpec((1, D), lambda i, j: (0, 0)),
            pl.BlockSpec((D, BC), lambda i, j: (0, j)),
            pl.BlockSpec((1, BC), lambda i, j: (0, j)),
        ],
        out_specs=pl.BlockSpec((BB, BC), lambda i, j: (i, j)),
        out_shape=jax.ShapeDtypeStruct((B, C), jnp.float32),
    )(x_proj, b_proj.reshape(1, D), wh_t, b_head.reshape(1, C))


def kernel(x, W_proj, b_proj, W_head, b_head):
    x_pad = jnp.pad(x.astype(jnp.int32), ((0, 0), (0, LP - L)),
                    constant_values=V)
    table = jnp.concatenate(
        [W_proj.T, jnp.zeros((TPAD - V, D), jnp.float32)], axis=0)
    x_proj = _sc_embed(x_pad, table)
    return _head_mm(x_proj, b_proj, W_head.T, b_head)


# speedup vs baseline: 4.5717x; 1.0129x over previous
import jax
import jax.numpy as jnp
from jax import lax
from jax.experimental import pallas as pl
from jax.experimental.pallas import tpu as pltpu
from jax.experimental.pallas import tpu_sc as plsc

B = 1024
V = 100000
D = 128
C = 5000
L = 100

LANES = 16
LP = 112
NV = LP // LANES
NC, NS = 2, 16
NW = NC * NS
RPT = B // NW
ND = D // LANES
CH = 5376
NCHUNK = 19
TPAD = NCHUNK * CH
SLICE = CH // NS
GCAP = 144


def _sc_embed_body(x_hbm, table_hbm, out_hbm,
                   xt, mark, glist0, glist1, grows0, grows1, accv,
                   sbuf0, sbuf1, semst0, semst1, semg0, semg1):
    cid = lax.axis_index("c")
    sid = lax.axis_index("s")
    wid = sid * NC + cid
    base = wid * RPT
    sbufs = (sbuf0, sbuf1)
    semsts = (semst0, semst1)

    pltpu.sync_copy(x_hbm.at[pl.ds(base, RPT)], xt)

    zf = jnp.zeros((LANES,), jnp.float32)

    def zero_row(r, carry):
        for d8 in range(ND):
            accv[r, pl.ds(d8 * LANES, LANES)] = zf
        return carry

    lax.fori_loop(0, RPT, zero_row, 0)

    def issue_stage(c, par):
        sb, sem = sbufs[par], semsts[par]
        row0 = c * CH + sid * SLICE
        pltpu.async_copy(table_hbm.at[pl.ds(row0, SLICE)],
                         sb.at[pl.ds(sid * SLICE, SLICE)], sem)

    def wait_stage(c, par):
        sb, sem = sbufs[par], semsts[par]
        row0 = c * CH + sid * SLICE
        pltpu.make_async_copy(table_hbm.at[pl.ds(row0, SLICE)],
                              sb.at[pl.ds(sid * SLICE, SLICE)], sem).wait()

    issue_stage(0, 0)
    issue_stage(1, 1)

    def chunk_body(c, par):
        sb = sbufs[par]
        lo = c * CH

        wait_stage(c, par)
        plsc.subcore_barrier()

        glists = (glist0, glist1)
        growss = (grows0, grows1)
        semgs = (semg0, semg1)

        def filter_fire(r, par, _sb=sb, _lo=lo):
            gl, gr, sg = glists[par], growss[par], semgs[par]
            zi = jnp.zeros((LANES,), jnp.int32)
            for q in range(8):
                gl[q, pl.ds(0, LANES)] = zi
            lidxs, masks, lanes = [], [], []
            for v in range(NV):
                idxv = xt[r, pl.ds(v * LANES, LANES)]
                lidx = idxv - _lo
                m = (lidx >= 0) & (lidx < jnp.int32(CH))
                lane = jnp.int32(v * LANES) + lax.iota(jnp.int32, LANES)
                plsc.store_scatter(mark, [lidx], lane, mask=m)
                lidxs.append(lidx)
                masks.append(m)
                lanes.append(lane)
            off = jnp.zeros((LANES,), jnp.int32)
            for v in range(NV):
                g = plsc.load_gather(mark, [lidxs[v]], mask=masks[v])
                keep = masks[v] & (g == lanes[v])
                cum = jnp.cumsum(keep.astype(jnp.int32))
                dest = off + cum - 1
                plsc.store_scatter(
                    gl, [jnp.right_shift(dest, 4), dest & jnp.int32(15)],
                    lidxs[v], mask=keep)
                off = off + plsc.all_reduce_population_count(keep)
            k = off[0]
            n16 = (k + 15) >> 4

            def fire(g2, c2):
                pltpu.async_copy(
                    _sb.at[gl.at[g2]],
                    gr.at[pl.ds(g2 * LANES, LANES)], sg)
                return c2

            lax.fori_loop(0, n16, fire, 0)
            return k

        def drain(par, k, _sb=sb):
            gl, gr, sg = glists[par], growss[par], semgs[par]
            n16 = (k + 15) >> 4

            def one(g2, c2):
                pltpu.make_async_copy(
                    _sb.at[gl.at[g2]],
                    gr.at[pl.ds(g2 * LANES, LANES)], sg).wait()
                return c2

            lax.fori_loop(0, n16, one, 0)

        def drain_acc(r, par, k):
            drain(par, k)
            gr = growss[par]
            acc = tuple(accv[r, pl.ds(d8 * LANES, LANES)]
                        for d8 in range(ND))

            def add_j(j, a):
                return tuple(a[d8] + gr[j, pl.ds(d8 * LANES, LANES)]
                             for d8 in range(ND))

            acc = lax.fori_loop(0, k, add_j, acc)
            for d8 in range(ND):
                accv[r, pl.ds(d8 * LANES, LANES)] = acc[d8]

        k_even = filter_fire(0, 0)

        def pair(t, kc):
            r0 = 2 * t
            k_odd = filter_fire(r0 + 1, 1)
            drain_acc(r0, 0, kc)
            k_next = filter_fire(jnp.minimum(r0 + 2, RPT - 1), 0)
            drain_acc(r0 + 1, 1, k_odd)
            return k_next

        k_left = lax.fori_loop(0, RPT // 2, pair, k_even)
        drain(0, k_left)
        plsc.subcore_barrier()

        @pl.when(c + 2 < NCHUNK)
        def _():
            issue_stage(c + 2, par)

    def chunk_pair(u, carry):
        chunk_body(2 * u, 0)
        chunk_body(2 * u + 1, 1)
        return carry

    lax.fori_loop(0, NCHUNK // 2, chunk_pair, 0)
    if NCHUNK % 2:
        chunk_body(jnp.int32(NCHUNK - 1), 0)

    pltpu.sync_copy(accv, out_hbm.at[pl.ds(base, RPT)])


def _sc_embed(x_pad, table):
    mesh = plsc.VectorSubcoreMesh(core_axis_name="c", subcore_axis_name="s")
    fn = pl.kernel(
        _sc_embed_body,
        mesh=mesh,
        compiler_params=pltpu.CompilerParams(needs_layout_passes=False),
        out_type=jax.ShapeDtypeStruct((B, D), jnp.float32),
        scratch_types=[
            pltpu.VMEM((RPT, LP), jnp.int32),
            pltpu.VMEM((CH,), jnp.int32),
            pltpu.VMEM((LP // LANES + 1, LANES), jnp.int32),
            pltpu.VMEM((LP // LANES + 1, LANES), jnp.int32),
            pltpu.VMEM((LP, D), jnp.float32),
            pltpu.VMEM((LP, D), jnp.float32),
            pltpu.VMEM((RPT, D), jnp.float32),
            pltpu.VMEM_SHARED((CH, D), jnp.float32),
            pltpu.VMEM_SHARED((CH, D), jnp.float32),
            pltpu.SemaphoreType.DMA,
            pltpu.SemaphoreType.DMA,
            pltpu.SemaphoreType.DMA,
            pltpu.SemaphoreType.DMA,
        ],
    )
    return fn(x_pad, table)


BB = 1024
BC = 2560


def _head_mm_body(xp_ref, bp_ref, wh_ref, bh_ref, o_ref):
    acc = jax.lax.dot_general(
        xp_ref[...] + bp_ref[...], wh_ref[...], (((1,), (0,)), ((), ())),
        preferred_element_type=jnp.float32)
    o_ref[...] = acc + bh_ref[...]


def _head_mm(x_proj, b_proj, wh_t, b_head):
    grid = (B // BB, pl.cdiv(C, BC))
    return pl.pallas_call(
        _head_mm_body,
        grid=grid,
        in_specs=[
            pl.BlockSpec((BB, D), lambda i, j: (i, 0)),
            pl.BlockSpec((1, D), lambda i, j: (0, 0)),
            pl.BlockSpec((D, BC), lambda i, j: (0, j)),
            pl.BlockSpec((1, BC), lambda i, j: (0, j)),
        ],
        out_specs=pl.BlockSpec((BB, BC), lambda i, j: (i, j)),
        out_shape=jax.ShapeDtypeStruct((B, C), jnp.float32),
    )(x_proj, b_proj.reshape(1, D), wh_t, b_head.reshape(1, C))


def kernel(x, W_proj, b_proj, W_head, b_head):
    x_pad = jnp.pad(x.astype(jnp.int32), ((0, 0), (0, LP - L)),
                    constant_values=V)
    table = jnp.concatenate(
        [W_proj.T, jnp.zeros((TPAD - V, D), jnp.float32)], axis=0)
    x_proj = _sc_embed(x_pad, table)
    return _head_mm(x_proj, b_proj, W_head.T, b_head)
